# baseline (speedup 1.0000x reference)
import functools

import jax
import jax.numpy as jnp
from jax import lax
from jax.experimental import pallas as pl
from jax.experimental.pallas import tpu as pltpu
from jax.experimental.pallas import tpu_sc as plsc

EMB = 64
EMB_PAD = 128
ROWS = 16384
COLS = 50
COLS_PAD = 56
B_TOTAL = ROWS * COLS
NUM_WORKERS = 32
PER_W = B_TOTAL // NUM_WORKERS
XR_PER_W = ROWS // NUM_WORKERS
CHUNK_XR = 16
CHUNK = CHUNK_XR * COLS
NCHUNK = XR_PER_W // CHUNK_XR


def _emb_body(x_hbm, tab_hbm, out_hbm,
              idx_all, rows0, rows1, gsem0, gsem1, wsem0, wsem1):
    wid = lax.axis_index("s") * 2 + lax.axis_index("c")
    base = wid * PER_W
    xr_base = wid * XR_PER_W
    rows = (rows0, rows1)
    gsem = (gsem0, gsem1)
    wsem = (wsem0, wsem1)

    pltpu.sync_copy(x_hbm.at[pl.ds(base, PER_W)], idx_all)

    def gather(i, b):
        pltpu.async_copy(
            tab_hbm.at[idx_all.at[pl.ds(i * CHUNK, CHUNK)]], rows[b], gsem[b])

    def wait_gather(i, b):
        pltpu.make_async_copy(
            tab_hbm.at[idx_all.at[pl.ds(i * CHUNK, CHUNK)]], rows[b],
            gsem[b]).wait()

    def writeout(i, b):
        for j in range(CHUNK_XR):
            pltpu.async_copy(
                rows[b].at[pl.ds(j * COLS, COLS)],
                out_hbm.at[xr_base + i * CHUNK_XR + j,
                           pl.ds(0, COLS), pl.ds(0, EMB)],
                wsem[b])

    def wait_writeout(i, b):
        for j in range(CHUNK_XR):
            pltpu.make_async_copy(
                rows[b].at[pl.ds(j * COLS, COLS)],
                out_hbm.at[xr_base + i * CHUNK_XR + j,
                           pl.ds(0, COLS), pl.ds(0, EMB)],
                wsem[b]).wait()

    gather(0, 0)
    gather(1, 1)

    def outer(g, carry):
        for b in range(2):
            i = 2 * g + b
            wait_gather(i, b)
            writeout(i, b)
            wait_writeout(i, b)
            gather(i + 2, b)
        return carry

    lax.fori_loop(0, NCHUNK // 2 - 1, outer, 0)

    for b in range(2):
        i = NCHUNK - 2 + b
        wait_gather(i, b)
        writeout(i, b)
    for b in range(2):
        wait_writeout(NCHUNK - 2 + b, b)


_emb = functools.partial(
    pl.kernel,
    out_type=jax.ShapeDtypeStruct((ROWS, COLS_PAD, EMB_PAD), jnp.float32),
    mesh=plsc.VectorSubcoreMesh(core_axis_name="c", subcore_axis_name="s"),
    scratch_types=[
        pltpu.VMEM((PER_W,), jnp.int32),
        pltpu.VMEM((CHUNK, EMB), jnp.float32),
        pltpu.VMEM((CHUNK, EMB), jnp.float32),
        pltpu.SemaphoreType.DMA,
        pltpu.SemaphoreType.DMA,
        pltpu.SemaphoreType.DMA,
        pltpu.SemaphoreType.DMA,
    ],
    compiler_params=pltpu.CompilerParams(use_tc_tiling_on_sc=False),
)(_emb_body)


def kernel(x, input_weight):
    flat = x.reshape(-1).astype(jnp.int32)
    out_big = _emb(flat, input_weight)
    return out_big[:, :COLS, :EMB] + input_weight[0]


# speedup vs baseline: 1.7202x; 1.7202x over previous
import functools

import jax
import jax.numpy as jnp
from jax import lax
from jax.experimental import pallas as pl
from jax.experimental.pallas import tpu as pltpu
from jax.experimental.pallas import tpu_sc as plsc

EMB = 64
EMB_PAD = 128
ROWS = 16384
COLS = 50
COLS_PAD = 56
B_TOTAL = ROWS * COLS
NUM_WORKERS = 32
PER_W = B_TOTAL // NUM_WORKERS
XR_PER_W = ROWS // NUM_WORKERS
CHUNK_XR = 16
CHUNK = CHUNK_XR * COLS
NCHUNK = XR_PER_W // CHUNK_XR


def _emb_body(x_hbm, tab_hbm, out_hbm,
              idx_all, rows0, rows1, gsem0, gsem1, wsem0, wsem1):
    wid = lax.axis_index("s") * 2 + lax.axis_index("c")
    base = wid * PER_W
    xr_base = wid * XR_PER_W
    rows = (rows0, rows1)
    gsem = (gsem0, gsem1)
    wsem = (wsem0, wsem1)

    pltpu.sync_copy(x_hbm.at[pl.ds(base, PER_W)], idx_all)

    def gather(i, b):
        pltpu.async_copy(
            tab_hbm.at[idx_all.at[pl.ds(i * CHUNK, CHUNK)]], rows[b], gsem[b])

    def wait_gather(i, b):
        pltpu.make_async_copy(
            tab_hbm.at[idx_all.at[pl.ds(i * CHUNK, CHUNK)]], rows[b],
            gsem[b]).wait()

    def writeout(i, b):
        for j in range(CHUNK_XR):
            pltpu.async_copy(
                rows[b].at[pl.ds(j * COLS, COLS)],
                out_hbm.at[xr_base + i * CHUNK_XR + j,
                           pl.ds(0, COLS), pl.ds(0, EMB)],
                wsem[b])

    def wait_writeout(i, b):
        for j in range(CHUNK_XR):
            pltpu.make_async_copy(
                rows[b].at[pl.ds(j * COLS, COLS)],
                out_hbm.at[xr_base + i * CHUNK_XR + j,
                           pl.ds(0, COLS), pl.ds(0, EMB)],
                wsem[b]).wait()

    gather(0, 0)
    gather(1, 1)

    def outer(g, carry):
        for b in range(2):
            i = 2 * g + b
            wait_gather(i, b)
            writeout(i, b)
            wait_writeout(i, b)
            gather(i + 2, b)
        return carry

    lax.fori_loop(0, NCHUNK // 2 - 1, outer, 0)

    for b in range(2):
        i = NCHUNK - 2 + b
        wait_gather(i, b)
        writeout(i, b)
    for b in range(2):
        wait_writeout(NCHUNK - 2 + b, b)


_emb = functools.partial(
    pl.kernel,
    out_type=jax.ShapeDtypeStruct((ROWS, COLS_PAD, EMB_PAD), jnp.float32),
    mesh=plsc.VectorSubcoreMesh(core_axis_name="c", subcore_axis_name="s"),
    scratch_types=[
        pltpu.VMEM((PER_W,), jnp.int32),
        pltpu.VMEM((CHUNK, EMB), jnp.float32),
        pltpu.VMEM((CHUNK, EMB), jnp.float32),
        pltpu.SemaphoreType.DMA,
        pltpu.SemaphoreType.DMA,
        pltpu.SemaphoreType.DMA,
        pltpu.SemaphoreType.DMA,
    ],
    compiler_params=pltpu.CompilerParams(use_tc_tiling_on_sc=False),
)(_emb_body)


def kernel(x, input_weight):
    flat = x.reshape(-1).astype(jnp.int32)
    out_big = _emb(flat, input_weight)
    return out_big[:, :COLS, :EMB]
